# Initial kernel scaffold; baseline (speedup 1.0000x reference)
import jax
import jax.numpy as jnp
from jax.experimental import pallas as pl


def kernel(species, edge_src, edge_dst, distances, switch, vec, angle_src, angle_dst, central_atom, W0, b0, W1, b1, W2, b2, W3, b3):
    raise NotImplementedError("write your pallas kernel here")


# speedup vs baseline: 7.4572x; 7.4572x over previous
import functools

import jax
import jax.numpy as jnp
from jax import lax
from jax.experimental import pallas as pl
from jax.experimental.pallas import tpu as pltpu
from jax.experimental.pallas import tpu_sc as plsc

N_NODES = 10000
N_EDGES = 160000
N_ANGLES = 320000
N_SPECIES = 16
DIM = 64

NC = 2
NS = 16
L = 16
NW = NC * NS

TBL_W = 8
E_PER_W = N_EDGES // NW

AP = 327680
AP_PER_W = AP // NW
B_CHUNK = 2048
B_ROWS = B_CHUNK // 128
B_NCHUNK = AP_PER_W // B_CHUNK

BA = 2560
NB = AP // BA

D_YCHUNK = 1024
D_NY = AP_PER_W // D_YCHUNK
N_STRIPE = 624
N_TAIL = N_NODES - NS * N_STRIPE

@functools.lru_cache(maxsize=None)
def _mesh():
    return plsc.VectorSubcoreMesh(
        core_axis_name="c", subcore_axis_name="s", num_cores=NC, num_subcores=NS)


_SC_PARAMS = pltpu.CompilerParams(needs_layout_passes=False,
                                  use_tc_tiling_on_sc=False)


def _wid():
    return lax.axis_index("s") * NC + lax.axis_index("c")


def _iota16():
    return lax.iota(jnp.int32, L)


@functools.lru_cache(maxsize=None)
def _edge_stage():
    return pl.kernel(
        _edge_body,
        out_type=jax.ShapeDtypeStruct((N_EDGES * TBL_W,), jnp.float32),
        mesh=_mesh(),
        compiler_params=_SC_PARAMS,
        scratch_types=[
            pltpu.VMEM((E_PER_W,), jnp.int32),
            pltpu.VMEM((E_PER_W,), jnp.float32),
            pltpu.VMEM((E_PER_W,), jnp.float32),
            pltpu.VMEM((E_PER_W * 3,), jnp.float32),
            pltpu.VMEM((N_NODES,), jnp.int32),
            pltpu.VMEM((E_PER_W * TBL_W,), jnp.float32),
        ],
    )


def _edge_body(edge_dst_h, dist_h, sw_h, vec_h, species_h, tbl_h,
               edst_v, d_v, sw_v, vec_v, spec_v, tbl_v):
    base = pl.multiple_of(_wid() * E_PER_W, E_PER_W)
    pltpu.sync_copy(edge_dst_h.at[pl.ds(base, E_PER_W)], edst_v)
    pltpu.sync_copy(dist_h.at[pl.ds(base, E_PER_W)], d_v)
    pltpu.sync_copy(sw_h.at[pl.ds(base, E_PER_W)], sw_v)
    pltpu.sync_copy(vec_h.at[pl.ds(base * 3, E_PER_W * 3)], vec_v)
    pltpu.sync_copy(species_h, spec_v)

    iot = _iota16()

    def lane(off):
        rows = off + iot
        d16 = d_v[pl.ds(off, L)]
        sw16 = sw_v[pl.ds(off, L)]
        r3 = rows * 3
        vx = plsc.load_gather(vec_v, [r3])
        vy = plsc.load_gather(vec_v, [r3 + 1])
        vz = plsc.load_gather(vec_v, [r3 + 2])
        edst16 = edst_v[pl.ds(off, L)]
        spec16 = plsc.load_gather(spec_v, [edst16])
        inv = 1.0 / d16
        sij = sw16 * inv
        f = sij * inv
        r8 = rows * TBL_W
        plsc.store_scatter(tbl_v, [r8], sij)
        plsc.store_scatter(tbl_v, [r8 + 1], f * vx)
        plsc.store_scatter(tbl_v, [r8 + 2], f * vy)
        plsc.store_scatter(tbl_v, [r8 + 3], f * vz)
        plsc.store_scatter(tbl_v, [r8 + 4], spec16.astype(jnp.float32))

    def body(j, carry):
        lane(j * L)
        return carry

    lax.fori_loop(0, E_PER_W // L, body, 0)
    lane(E_PER_W - L)
    pltpu.sync_copy(tbl_v, tbl_h.at[pl.ds(base * TBL_W, E_PER_W * TBL_W)])


@functools.lru_cache(maxsize=None)
def _angle_stage():
    return pl.kernel(
        _angle_body,
        out_type=(jax.ShapeDtypeStruct((AP, TBL_W), jnp.float32),
                  jax.ShapeDtypeStruct((AP, TBL_W), jnp.float32)),
        mesh=_mesh(),
        compiler_params=_SC_PARAMS,
        scratch_types=[
            pltpu.VMEM((B_ROWS, 128), jnp.int32),
            pltpu.VMEM((B_ROWS, 128), jnp.int32),
            pltpu.VMEM((B_CHUNK, TBL_W), jnp.float32),
            pltpu.VMEM((B_CHUNK, TBL_W), jnp.float32),
            pltpu.SemaphoreType.DMA,
        ],
    )


def _angle_body(tbl_h, asrc_h, adst_h, ga_h, gb_h,
                asrc_v, adst_v, ra_v, rb_v, sem):
    wid = _wid()

    def chunk(t, carry):
        cb = pl.multiple_of(wid * AP_PER_W + t * B_CHUNK, B_CHUNK)
        rowb = pl.multiple_of(cb // 128, B_ROWS)
        pltpu.sync_copy(asrc_h.at[pl.ds(rowb, B_ROWS)], asrc_v)
        pltpu.sync_copy(adst_h.at[pl.ds(rowb, B_ROWS)], adst_v)
        descs = []
        for k in range(B_ROWS):
            descs.append(pltpu.async_copy(
                tbl_h.at[asrc_v.at[k]], ra_v.at[pl.ds(k * 128, 128)], sem))
        for k in range(B_ROWS):
            descs.append(pltpu.async_copy(
                tbl_h.at[adst_v.at[k]], rb_v.at[pl.ds(k * 128, 128)], sem))
        for dsc in descs:
            dsc.wait()
        pltpu.sync_copy(ra_v, ga_h.at[pl.ds(cb, B_CHUNK)])
        pltpu.sync_copy(rb_v, gb_h.at[pl.ds(cb, B_CHUNK)])
        return carry

    lax.fori_loop(0, B_NCHUNK, chunk, 0)


def _silu(x):
    return x / (1.0 + jnp.exp(-x))


def _mlp_block(ga_ref, gb_ref, w0_ref, b0_ref, w1_ref, b1_ref,
               w2_ref, b2_ref, w3_ref, b3_ref, y_ref):
    ga = ga_ref[...]
    gb = gb_ref[...]
    th = jnp.sum(ga[:, 0:4] * gb[:, 0:4], axis=1, keepdims=True)
    glob = pl.program_id(0) * BA + lax.broadcasted_iota(jnp.int32, (BA, 1), 0)
    th = jnp.where(glob < N_ANGLES, th, 0.0)
    code = (ga[:, 4:5].astype(jnp.int32) * N_SPECIES
            + gb[:, 4:5].astype(jnp.int32))
    w0 = w0_ref[...]
    u = w0[0:1, :]
    ar = w0[1:1 + N_SPECIES, :]
    br = w0[1 + N_SPECIES:1 + 2 * N_SPECIES, :]
    b0 = b0_ref[...]

    p_iota = lax.broadcasted_iota(jnp.int32, (256, N_SPECIES), 0)
    s_sel = (p_iota // N_SPECIES ==
             lax.broadcasted_iota(jnp.int32, (256, N_SPECIES), 1)).astype(jnp.float32)
    d_sel = (p_iota % N_SPECIES ==
             lax.broadcasted_iota(jnp.int32, (256, N_SPECIES), 1)).astype(jnp.float32)
    sa = jnp.dot(s_sel, ar, preferred_element_type=jnp.float32)
    sb = jnp.dot(s_sel, br, preferred_element_type=jnp.float32)
    da = jnp.dot(d_sel, ar, preferred_element_type=jnp.float32)
    db = jnp.dot(d_sel, br, preferred_element_type=jnp.float32)
    cboth = jnp.concatenate([sa + db + b0, sb + da + b0], axis=1)

    oh = (code ==
          lax.broadcasted_iota(jnp.int32, (BA, 256), 1)).astype(jnp.float32)
    u2 = jnp.concatenate([u, u], axis=1)
    x = jnp.dot(oh, cboth, preferred_element_type=jnp.float32)
    x = x + th * u2
    h = _silu(x)

    z64 = jnp.zeros((DIM, DIM), jnp.float32)
    for w_ref, b_ref in ((w1_ref, b1_ref), (w2_ref, b2_ref)):
        w = w_ref[...]
        b = b_ref[...]
        wd = jnp.concatenate([
            jnp.concatenate([w, z64], axis=1),
            jnp.concatenate([z64, w], axis=1)], axis=0)
        bd = jnp.concatenate([b, b], axis=1)
        h = _silu(jnp.dot(h, wd, preferred_element_type=jnp.float32) + bd)

    w3c = jnp.concatenate([w3_ref[...], w3_ref[...]], axis=0)
    g = jnp.dot(h, w3c, preferred_element_type=jnp.float32) + 2.0 * b3_ref[...]
    y_ref[...] = g * th


def _mlp_stage(ga, gb, W0, b0, W1, b1, W2, b2, W3, b3):
    full = lambda shape: pl.BlockSpec(shape, lambda i: (0,) * len(shape))
    return pl.pallas_call(
        _mlp_block,
        grid=(NB,),
        in_specs=[
            pl.BlockSpec((BA, TBL_W), lambda i: (i, 0)),
            pl.BlockSpec((BA, TBL_W), lambda i: (i, 0)),
            full((1 + 2 * N_SPECIES, DIM)), full((1, DIM)),
            full((DIM, DIM)), full((1, DIM)),
            full((DIM, DIM)), full((1, DIM)),
            full((DIM, DIM)), full((1, DIM)),
        ],
        out_specs=pl.BlockSpec((BA, DIM), lambda i: (i, 0)),
        out_shape=jax.ShapeDtypeStruct((AP, DIM), jnp.float32),
    )(ga, gb, W0, b0, W1, b1, W2, b2, W3, b3)


@functools.lru_cache(maxsize=None)
def _scatter_stage():
    return pl.kernel(
        _scatter_body,
        out_type=jax.ShapeDtypeStruct((NC, N_NODES, DIM), jnp.float32),
        mesh=_mesh(),
        compiler_params=_SC_PARAMS,
        scratch_types=[
            pltpu.VMEM((D_YCHUNK, DIM), jnp.float32),
            pltpu.VMEM((B_ROWS, 128), jnp.int32),
            pltpu.VMEM_SHARED((N_NODES, DIM), jnp.float32),
        ],
    )


def _scatter_body(y_h, cen_h, zeros_h, out_h, y_v, cen_v, acc_sh):
    c = lax.axis_index("c")
    s = lax.axis_index("s")
    wid = s * NC + c
    stripe = pl.multiple_of(s * N_STRIPE, N_STRIPE)
    pltpu.sync_copy(zeros_h.at[pl.ds(stripe, N_STRIPE)],
                    acc_sh.at[pl.ds(stripe, N_STRIPE)])

    @pl.when(s == NS - 1)
    def _():
        pltpu.sync_copy(zeros_h.at[pl.ds(NS * N_STRIPE, N_TAIL)],
                        acc_sh.at[pl.ds(NS * N_STRIPE, N_TAIL)])

    plsc.subcore_barrier()

    def ychunk(t, carry):
        yb = pl.multiple_of(wid * AP_PER_W + t * D_YCHUNK, D_YCHUNK)
        pltpu.sync_copy(y_h.at[pl.ds(yb, D_YCHUNK)], y_v)
        pltpu.sync_copy(cen_h.at[pl.ds(pl.multiple_of(yb // 128, 8),
                                       D_YCHUNK // 128)],
                        cen_v.at[pl.ds(0, D_YCHUNK // 128)])
        for k in range(D_YCHUNK // 128):
            pltpu.sync_copy(y_v.at[pl.ds(k * 128, 128)],
                            acc_sh.at[cen_v.at[k]], add=True)
        return carry

    lax.fori_loop(0, D_NY, ychunk, 0)
    plsc.subcore_barrier()
    pltpu.sync_copy(acc_sh.at[pl.ds(stripe, N_STRIPE)],
                    out_h.at[c, pl.ds(stripe, N_STRIPE)])

    @pl.when(s == NS - 1)
    def _():
        pltpu.sync_copy(acc_sh.at[pl.ds(NS * N_STRIPE, N_TAIL)],
                        out_h.at[c, pl.ds(NS * N_STRIPE, N_TAIL)])


def _sum_block(a_ref, o_ref):
    o_ref[...] = a_ref[0] + a_ref[1]


def _sum_stage(acc2):
    return pl.pallas_call(
        _sum_block,
        grid=(5,),
        in_specs=[pl.BlockSpec((NC, 2000, DIM), lambda i: (0, i, 0))],
        out_specs=pl.BlockSpec((2000, DIM), lambda i: (i, 0)),
        out_shape=jax.ShapeDtypeStruct((N_NODES, DIM), jnp.float32),
    )(acc2)


def kernel(species, edge_src, edge_dst, distances, switch, vec,
           angle_src, angle_dst, central_atom,
           W0, b0, W1, b1, W2, b2, W3, b3):
    del edge_src
    species = species.astype(jnp.int32)
    edge_dst = edge_dst.astype(jnp.int32)
    pad = AP - N_ANGLES
    padi = lambda a: jnp.concatenate(
        [a.astype(jnp.int32), jnp.zeros((pad,), jnp.int32)]).reshape(AP // 128, 128)
    asrc2 = padi(angle_src)
    adst2 = padi(angle_dst)
    cen2 = padi(central_atom)

    tbl = _edge_stage()(edge_dst, distances, switch,
                        vec.reshape(N_EDGES * 3), species)
    ga, gb = _angle_stage()(tbl.reshape(N_EDGES, TBL_W), asrc2, adst2)
    y = _mlp_stage(ga, gb,
                   W0, b0.reshape(1, DIM), W1, b1.reshape(1, DIM),
                   W2, b2.reshape(1, DIM), W3, b3.reshape(1, DIM))
    acc2 = _scatter_stage()(y, cen2, jnp.zeros((N_NODES, DIM), jnp.float32))
    return _sum_stage(acc2)


# speedup vs baseline: 7.8569x; 1.0536x over previous
import functools

import jax
import jax.numpy as jnp
from jax import lax
from jax.experimental import pallas as pl
from jax.experimental.pallas import tpu as pltpu
from jax.experimental.pallas import tpu_sc as plsc

N_NODES = 10000
N_EDGES = 160000
N_ANGLES = 320000
N_SPECIES = 16
DIM = 64

NC = 2
NS = 16
L = 16
NW = NC * NS

TBL_W = 8
E_PER_W = N_EDGES // NW

A_ROWS = N_ANGLES // 128
W_ROWS = 80
T_ROWS = A_ROWS - 31 * W_ROWS
B_ROWS = 16
B_CHUNK = B_ROWS * 128
T_FULL = T_ROWS // B_ROWS
T_REM = T_ROWS - T_FULL * B_ROWS

BA = 2560
NB = N_ANGLES // BA

D_ROWS = 8
D_CHUNK = D_ROWS * 128
D_FULL_W = W_ROWS // D_ROWS
D_FULL_T = T_ROWS // D_ROWS
D_REM = T_ROWS - D_FULL_T * D_ROWS

N_STRIPE = 624
N_TAIL = N_NODES - NS * N_STRIPE

@functools.lru_cache(maxsize=None)
def _mesh():
    return plsc.VectorSubcoreMesh(
        core_axis_name="c", subcore_axis_name="s", num_cores=NC, num_subcores=NS)


_SC_PARAMS = pltpu.CompilerParams(needs_layout_passes=False,
                                  use_tc_tiling_on_sc=False)


def _wid():
    return lax.axis_index("s") * NC + lax.axis_index("c")


@functools.lru_cache(maxsize=None)
def _edge_stage():
    return pl.kernel(
        _edge_body,
        out_type=jax.ShapeDtypeStruct((N_EDGES, TBL_W), jnp.float32),
        mesh=_mesh(),
        compiler_params=_SC_PARAMS,
        scratch_types=[
            pltpu.VMEM((E_PER_W,), jnp.int32),
            pltpu.VMEM((E_PER_W,), jnp.float32),
            pltpu.VMEM((E_PER_W,), jnp.float32),
            pltpu.VMEM((E_PER_W, 3), jnp.float32),
            pltpu.VMEM((N_NODES,), jnp.int32),
            pltpu.VMEM((E_PER_W, TBL_W), jnp.float32),
        ],
    )


def _edge_body(edge_dst_h, dist_h, sw_h, vec_h, species_h, tbl_h,
               edst_v, d_v, sw_v, vec_v, spec_v, tbl_v):
    base = pl.multiple_of(_wid() * E_PER_W, E_PER_W)
    pltpu.sync_copy(edge_dst_h.at[pl.ds(base, E_PER_W)], edst_v)
    pltpu.sync_copy(dist_h.at[pl.ds(base, E_PER_W)], d_v)
    pltpu.sync_copy(sw_h.at[pl.ds(base, E_PER_W)], sw_v)
    pltpu.sync_copy(vec_h.at[pl.ds(base, E_PER_W)], vec_v)
    pltpu.sync_copy(species_h, spec_v)

    iot = lax.iota(jnp.int32, L)
    c0 = jnp.full((L,), 0, jnp.int32)
    c1 = jnp.full((L,), 1, jnp.int32)
    c2 = jnp.full((L,), 2, jnp.int32)
    c3 = jnp.full((L,), 3, jnp.int32)
    c4 = jnp.full((L,), 4, jnp.int32)

    def lane(off):
        rows = off + iot
        d16 = d_v[pl.ds(off, L)]
        sw16 = sw_v[pl.ds(off, L)]
        vx = plsc.load_gather(vec_v, [rows, c0])
        vy = plsc.load_gather(vec_v, [rows, c1])
        vz = plsc.load_gather(vec_v, [rows, c2])
        edst16 = edst_v[pl.ds(off, L)]
        spec16 = plsc.load_gather(spec_v, [edst16])
        inv = 1.0 / d16
        sij = sw16 * inv
        f = sij * inv
        plsc.store_scatter(tbl_v, [rows, c0], sij)
        plsc.store_scatter(tbl_v, [rows, c1], f * vx)
        plsc.store_scatter(tbl_v, [rows, c2], f * vy)
        plsc.store_scatter(tbl_v, [rows, c3], f * vz)
        plsc.store_scatter(tbl_v, [rows, c4], spec16.astype(jnp.float32))

    def body(j, carry):
        lane(j * L)
        return carry

    lax.fori_loop(0, E_PER_W // L, body, 0)
    lane(E_PER_W - L)
    pltpu.sync_copy(tbl_v, tbl_h.at[pl.ds(base, E_PER_W)])


@functools.lru_cache(maxsize=None)
def _angle_stage():
    return pl.kernel(
        _angle_body,
        out_type=(jax.ShapeDtypeStruct((N_ANGLES, TBL_W), jnp.float32),
                  jax.ShapeDtypeStruct((N_ANGLES, TBL_W), jnp.float32)),
        mesh=_mesh(),
        compiler_params=_SC_PARAMS,
        scratch_types=[
            pltpu.VMEM((B_ROWS, 128), jnp.int32),
            pltpu.VMEM((B_ROWS, 128), jnp.int32),
            pltpu.VMEM((B_CHUNK, TBL_W), jnp.float32),
            pltpu.VMEM((B_CHUNK, TBL_W), jnp.float32),
            pltpu.SemaphoreType.DMA,
        ],
    )


def _gather_chunk(tbl_h, asrc_v, adst_v, ra_v, rb_v, sem, nrows):
    descs = []
    for k in range(nrows):
        descs.append(pltpu.async_copy(
            tbl_h.at[asrc_v.at[k]], ra_v.at[pl.ds(k * 128, 128)], sem))
    for k in range(nrows):
        descs.append(pltpu.async_copy(
            tbl_h.at[adst_v.at[k]], rb_v.at[pl.ds(k * 128, 128)], sem))
    for dsc in descs:
        dsc.wait()


def _angle_body(tbl_h, asrc_h, adst_h, ga_h, gb_h,
                asrc_v, adst_v, ra_v, rb_v, sem):
    wid = _wid()
    r0 = wid * W_ROWS

    def chunk(t, carry):
        rowb = pl.multiple_of(r0 + t * B_ROWS, B_ROWS)
        cb = pl.multiple_of(rowb * 128, B_CHUNK)
        pltpu.sync_copy(asrc_h.at[pl.ds(rowb, B_ROWS)], asrc_v)
        pltpu.sync_copy(adst_h.at[pl.ds(rowb, B_ROWS)], adst_v)
        _gather_chunk(tbl_h, asrc_v, adst_v, ra_v, rb_v, sem, B_ROWS)
        pltpu.sync_copy(ra_v, ga_h.at[pl.ds(cb, B_CHUNK)])
        pltpu.sync_copy(rb_v, gb_h.at[pl.ds(cb, B_CHUNK)])
        return carry

    nch = jnp.where(wid == NW - 1, T_FULL, W_ROWS // B_ROWS)
    lax.fori_loop(0, nch, chunk, 0)

    @pl.when(wid == NW - 1)
    def _():
        rowb = 31 * W_ROWS + T_FULL * B_ROWS
        cb = rowb * 128
        pltpu.sync_copy(asrc_h.at[pl.ds(rowb, T_REM)],
                        asrc_v.at[pl.ds(0, T_REM)])
        pltpu.sync_copy(adst_h.at[pl.ds(rowb, T_REM)],
                        adst_v.at[pl.ds(0, T_REM)])
        _gather_chunk(tbl_h, asrc_v, adst_v, ra_v, rb_v, sem, T_REM)
        pltpu.sync_copy(ra_v.at[pl.ds(0, T_REM * 128)],
                        ga_h.at[pl.ds(cb, T_REM * 128)])
        pltpu.sync_copy(rb_v.at[pl.ds(0, T_REM * 128)],
                        gb_h.at[pl.ds(cb, T_REM * 128)])


def _silu(x):
    return x / (1.0 + jnp.exp2(x * (-1.4426950408889634)))


def _mlp_block(ga_ref, gb_ref, w0_ref, b0_ref, w1_ref, b1_ref,
               w2_ref, b2_ref, w3_ref, b3_ref, y_ref):
    ga = ga_ref[...]
    gb = gb_ref[...]
    th = jnp.sum(ga[:, 0:4] * gb[:, 0:4], axis=1, keepdims=True)
    cs = ga[:, 4:5]
    cd = gb[:, 4:5]
    w0 = w0_ref[...]
    u = w0[0:1, :]
    ar = w0[1:1 + N_SPECIES, :]
    br = w0[1 + N_SPECIES:1 + 2 * N_SPECIES, :]
    b0 = b0_ref[...]

    p_iota = lax.broadcasted_iota(jnp.int32, (256, N_SPECIES), 0)
    s_sel = (p_iota // N_SPECIES ==
             lax.broadcasted_iota(jnp.int32, (256, N_SPECIES), 1)).astype(jnp.float32)
    d_sel = (p_iota % N_SPECIES ==
             lax.broadcasted_iota(jnp.int32, (256, N_SPECIES), 1)).astype(jnp.float32)
    sa = jnp.dot(s_sel, ar, preferred_element_type=jnp.float32)
    sb = jnp.dot(s_sel, br, preferred_element_type=jnp.float32)
    da = jnp.dot(d_sel, ar, preferred_element_type=jnp.float32)
    db = jnp.dot(d_sel, br, preferred_element_type=jnp.float32)
    cboth = jnp.concatenate([sa + db + b0, sb + da + b0], axis=1)

    code = cs * float(N_SPECIES) + cd
    oh = (code.astype(jnp.int32) ==
          lax.broadcasted_iota(jnp.int32, (BA, 256), 1)).astype(jnp.float32)
    u2 = jnp.concatenate([u, u], axis=1)
    x = jnp.dot(oh, cboth, preferred_element_type=jnp.float32) + th * u2
    h = _silu(x)

    z64 = jnp.zeros((DIM, DIM), jnp.float32)
    for w_ref, b_ref in ((w1_ref, b1_ref), (w2_ref, b2_ref)):
        w = w_ref[...]
        b = b_ref[...]
        wd = jnp.concatenate([
            jnp.concatenate([w, z64], axis=1),
            jnp.concatenate([z64, w], axis=1)], axis=0)
        bd = jnp.concatenate([b, b], axis=1)
        h = _silu(jnp.dot(h, wd, preferred_element_type=jnp.float32) + bd)

    w3c = jnp.concatenate([w3_ref[...], w3_ref[...]], axis=0)
    g = jnp.dot(h, w3c, preferred_element_type=jnp.float32) + 2.0 * b3_ref[...]
    y_ref[...] = g * th


def _mlp_stage(ga, gb, W0, b0, W1, b1, W2, b2, W3, b3):
    full = lambda shape: pl.BlockSpec(shape, lambda i: (0,) * len(shape))
    return pl.pallas_call(
        _mlp_block,
        grid=(NB,),
        in_specs=[
            pl.BlockSpec((BA, TBL_W), lambda i: (i, 0)),
            pl.BlockSpec((BA, TBL_W), lambda i: (i, 0)),
            full((1 + 2 * N_SPECIES, DIM)), full((1, DIM)),
            full((DIM, DIM)), full((1, DIM)),
            full((DIM, DIM)), full((1, DIM)),
            full((DIM, DIM)), full((1, DIM)),
        ],
        out_specs=pl.BlockSpec((BA, DIM), lambda i: (i, 0)),
        out_shape=jax.ShapeDtypeStruct((N_ANGLES, DIM), jnp.float32),
    )(ga, gb, W0, b0, W1, b1, W2, b2, W3, b3)


@functools.lru_cache(maxsize=None)
def _scatter_stage():
    return pl.kernel(
        _scatter_body,
        out_type=jax.ShapeDtypeStruct((NC, N_NODES, DIM), jnp.float32),
        mesh=_mesh(),
        compiler_params=_SC_PARAMS,
        scratch_types=[
            pltpu.VMEM((D_CHUNK, DIM), jnp.float32),
            pltpu.VMEM((D_ROWS, 128), jnp.int32),
            pltpu.VMEM_SHARED((N_NODES, DIM), jnp.float32),
        ],
    )


def _scatter_body(y_h, cen_h, zeros_h, out_h, y_v, cen_v, acc_sh):
    c = lax.axis_index("c")
    s = lax.axis_index("s")
    wid = s * NC + c
    stripe = pl.multiple_of(s * N_STRIPE, N_STRIPE)
    pltpu.sync_copy(zeros_h.at[pl.ds(stripe, N_STRIPE)],
                    acc_sh.at[pl.ds(stripe, N_STRIPE)])

    @pl.when(s == NS - 1)
    def _():
        pltpu.sync_copy(zeros_h.at[pl.ds(NS * N_STRIPE, N_TAIL)],
                        acc_sh.at[pl.ds(NS * N_STRIPE, N_TAIL)])

    plsc.subcore_barrier()

    r0 = wid * W_ROWS

    def ychunk(t, carry):
        rowb = pl.multiple_of(r0 + t * D_ROWS, D_ROWS)
        yb = pl.multiple_of(rowb * 128, D_CHUNK)
        pltpu.sync_copy(y_h.at[pl.ds(yb, D_CHUNK)], y_v)
        pltpu.sync_copy(cen_h.at[pl.ds(rowb, D_ROWS)], cen_v)
        for k in range(D_ROWS):
            pltpu.sync_copy(y_v.at[pl.ds(k * 128, 128)],
                            acc_sh.at[cen_v.at[k]], add=True)
        return carry

    nch = jnp.where(wid == NW - 1, D_FULL_T, D_FULL_W)
    lax.fori_loop(0, nch, ychunk, 0)

    @pl.when(wid == NW - 1)
    def _():
        rowb = 31 * W_ROWS + D_FULL_T * D_ROWS
        yb = rowb * 128
        pltpu.sync_copy(y_h.at[pl.ds(yb, D_REM * 128)],
                        y_v.at[pl.ds(0, D_REM * 128)])
        pltpu.sync_copy(cen_h.at[pl.ds(rowb, D_REM)],
                        cen_v.at[pl.ds(0, D_REM)])
        for k in range(D_REM):
            pltpu.sync_copy(y_v.at[pl.ds(k * 128, 128)],
                            acc_sh.at[cen_v.at[k]], add=True)

    plsc.subcore_barrier()
    pltpu.sync_copy(acc_sh.at[pl.ds(stripe, N_STRIPE)],
                    out_h.at[c, pl.ds(stripe, N_STRIPE)])

    @pl.when(s == NS - 1)
    def _():
        pltpu.sync_copy(acc_sh.at[pl.ds(NS * N_STRIPE, N_TAIL)],
                        out_h.at[c, pl.ds(NS * N_STRIPE, N_TAIL)])


def _sum_block(a_ref, o_ref):
    o_ref[...] = a_ref[0] + a_ref[1]


def _sum_stage(acc2):
    return pl.pallas_call(
        _sum_block,
        grid=(5,),
        in_specs=[pl.BlockSpec((NC, 2000, DIM), lambda i: (0, i, 0))],
        out_specs=pl.BlockSpec((2000, DIM), lambda i: (i, 0)),
        out_shape=jax.ShapeDtypeStruct((N_NODES, DIM), jnp.float32),
    )(acc2)


def kernel(species, edge_src, edge_dst, distances, switch, vec,
           angle_src, angle_dst, central_atom,
           W0, b0, W1, b1, W2, b2, W3, b3):
    del edge_src
    species = species.astype(jnp.int32)
    edge_dst = edge_dst.astype(jnp.int32)
    asrc2 = angle_src.astype(jnp.int32).reshape(A_ROWS, 128)
    adst2 = angle_dst.astype(jnp.int32).reshape(A_ROWS, 128)
    cen2 = central_atom.astype(jnp.int32).reshape(A_ROWS, 128)

    tbl = _edge_stage()(edge_dst, distances, switch, vec, species)
    ga, gb = _angle_stage()(tbl, asrc2, adst2)
    y = _mlp_stage(ga, gb,
                   W0, b0.reshape(1, DIM), W1, b1.reshape(1, DIM),
                   W2, b2.reshape(1, DIM), W3, b3.reshape(1, DIM))
    acc2 = _scatter_stage()(y, cen2, jnp.zeros((N_NODES, DIM), jnp.float32))
    return _sum_stage(acc2)
